# baseline (speedup 1.0000x reference)
import functools

import jax
import jax.numpy as jnp
from jax.experimental import pallas as pl
from jax.experimental.pallas import tpu as pltpu

_C = 768
_K = _C // 10


def _entropy_body(x_ref, out_ref):
    x = x_ref[0]
    g = jax.lax.dot_general(
        x, x, dimension_numbers=(((1,), (1,)), ((), ())),
        preferred_element_type=jnp.float32)
    xx = jnp.sum(x * x, axis=1)
    d2 = xx[:, None] + xx[None, :] - 2.0 * g
    d2 = jnp.maximum(d2, 1e-8)
    bits = jax.lax.bitcast_convert_type(d2, jnp.int32)
    row_i = jax.lax.broadcasted_iota(jnp.int32, (_C, _C), 0)
    col_i = jax.lax.broadcasted_iota(jnp.int32, (_C, _C), 1)
    off_diag = jnp.where(row_i == col_i, jnp.int32(0x7FFFFFFF), bits)
    lo0 = jnp.min(off_diag, axis=0, keepdims=True)
    hi0 = jnp.max(bits, axis=0, keepdims=True)

    def cond(carry):
        lo, hi = carry
        return jnp.any(lo < hi)

    def one_step(lo, hi):
        mid = lo + (hi - lo) // 2
        cnt = jnp.sum((bits <= mid).astype(jnp.int32), axis=0, keepdims=True)
        take_lo = cnt >= (_K + 1)
        hi = jnp.where(take_lo, mid, hi)
        lo = jnp.where(take_lo, lo, mid + 1)
        return lo, hi

    def step(carry):
        lo, hi = carry
        for _ in range(4):
            lo, hi = one_step(lo, hi)
        return lo, hi

    lo, _ = jax.lax.while_loop(cond, step, (lo0, hi0))
    kth = jax.lax.bitcast_convert_type(lo, jnp.float32)
    r_ball = jnp.sqrt(kth)
    out_ref[0] = jnp.full((1, 128), jnp.sum(r_ball), jnp.float32)


def _entropy_sum(feat):
    B, C, H, W = feat.shape
    x = feat.reshape(B, C, H * W)
    sums = pl.pallas_call(
        _entropy_body,
        grid=(B,),
        in_specs=[pl.BlockSpec((1, C, H * W), lambda i: (i, 0, 0))],
        out_specs=pl.BlockSpec((1, 1, 128), lambda i: (i, 0, 0)),
        out_shape=jax.ShapeDtypeStruct((B, 1, 128), jnp.float32),
        compiler_params=pltpu.CompilerParams(
            dimension_semantics=("parallel",)),
    )(x)
    return jnp.sum(sums[:, 0, 0])


@functools.partial(jax.jit, static_argnums=())
def kernel(feat0, feat1, feat2):
    h_total = jnp.stack([_entropy_sum(f) for f in (feat0, feat1, feat2)])
    ent = jnp.log(h_total + 1.0)
    delta = jnp.stack([ent[1] - ent[0], ent[2] - ent[1]])
    return jnp.var(delta, ddof=1)


# speedup vs baseline: 1.1342x; 1.1342x over previous
import functools

import jax
import jax.numpy as jnp
from jax.experimental import pallas as pl
from jax.experimental.pallas import tpu as pltpu

_C = 768
_K = _C // 10


def _entropy_body(x0_ref, x1_ref, x2_ref, out_ref, hsum_ref):
    b = pl.program_id(0)
    f = pl.program_id(1)
    x = jnp.where(f == 0, x0_ref[0],
                  jnp.where(f == 1, x1_ref[0], x2_ref[0]))
    g = jax.lax.dot_general(
        x, x, dimension_numbers=(((1,), (1,)), ((), ())),
        preferred_element_type=jnp.float32)
    xx = jnp.sum(x * x, axis=1)
    d2 = xx[:, None] + xx[None, :] - 2.0 * g
    d2 = jnp.maximum(d2, 1e-8)
    bits = jax.lax.bitcast_convert_type(d2, jnp.int32)
    row_i = jax.lax.broadcasted_iota(jnp.int32, (_C, _C), 0)
    col_i = jax.lax.broadcasted_iota(jnp.int32, (_C, _C), 1)
    off_diag = jnp.where(row_i == col_i, jnp.int32(0x7FFFFFFF), bits)
    lo0 = jnp.min(off_diag, axis=0, keepdims=True)
    hi0 = jnp.max(bits, axis=0, keepdims=True)

    def cond(carry):
        lo, hi = carry
        return jnp.any(lo < hi)

    def one_step(lo, hi):
        mid = lo + (hi - lo) // 2
        cnt = jnp.sum((bits <= mid).astype(jnp.int32), axis=0, keepdims=True)
        take_lo = cnt >= (_K + 1)
        hi = jnp.where(take_lo, mid, hi)
        lo = jnp.where(take_lo, lo, mid + 1)
        return lo, hi

    def step(carry):
        lo, hi = carry
        for _ in range(4):
            lo, hi = one_step(lo, hi)
        return lo, hi

    lo, _ = jax.lax.while_loop(cond, step, (lo0, hi0))
    kth = jax.lax.bitcast_convert_type(lo, jnp.float32)
    s = jnp.sum(jnp.sqrt(kth))

    @pl.when(b == 0)
    def _():
        hsum_ref[f] = s

    @pl.when(b != 0)
    def _():
        hsum_ref[f] = hsum_ref[f] + s

    @pl.when((b == pl.num_programs(0) - 1) & (f == pl.num_programs(1) - 1))
    def _():
        e0 = jnp.log(jnp.full((1, 128), hsum_ref[0]) + 1.0)
        e1 = jnp.log(jnp.full((1, 128), hsum_ref[1]) + 1.0)
        e2 = jnp.log(jnp.full((1, 128), hsum_ref[2]) + 1.0)
        d0 = e1 - e0
        d1 = e2 - e1
        out_ref[0] = (d0 - d1) * (d0 - d1) * 0.5


@functools.partial(jax.jit, static_argnums=())
def kernel(feat0, feat1, feat2):
    B, C, H, W = feat0.shape
    xs = [f.reshape(B, C, H * W) for f in (feat0, feat1, feat2)]
    out = pl.pallas_call(
        _entropy_body,
        grid=(B, 3),
        in_specs=[pl.BlockSpec((1, C, H * W), lambda b, f: (b, 0, 0))] * 3,
        out_specs=pl.BlockSpec((1, 1, 128), lambda b, f: (0, 0, 0)),
        out_shape=jax.ShapeDtypeStruct((1, 1, 128), jnp.float32),
        scratch_shapes=[pltpu.SMEM((4,), jnp.float32)],
        compiler_params=pltpu.CompilerParams(
            dimension_semantics=("arbitrary", "arbitrary")),
    )(*xs)
    return out[0, 0, 0]


# speedup vs baseline: 1.2354x; 1.0892x over previous
import functools

import jax
import jax.numpy as jnp
from jax.experimental import pallas as pl
from jax.experimental.pallas import tpu as pltpu

_C = 768
_K = _C // 10


def _bits_and_bracket(x):
    g = jax.lax.dot_general(
        x, x, dimension_numbers=(((1,), (1,)), ((), ())),
        preferred_element_type=jnp.float32)
    xx = jnp.sum(x * x, axis=1)
    d2 = xx[:, None] + xx[None, :] - 2.0 * g
    d2 = jnp.maximum(d2, 1e-8)
    bits = jax.lax.bitcast_convert_type(d2, jnp.int32)
    row_i = jax.lax.broadcasted_iota(jnp.int32, (_C, _C), 0)
    col_i = jax.lax.broadcasted_iota(jnp.int32, (_C, _C), 1)
    off_diag = jnp.where(row_i == col_i, jnp.int32(0x7FFFFFFF), bits)
    lo0 = jnp.min(off_diag, axis=0, keepdims=True)
    hi0 = jnp.max(bits, axis=0, keepdims=True)
    return bits, lo0, hi0


def _one_step(bits, lo, hi):
    mid = lo + (hi - lo) // 2
    cnt = jnp.sum((bits <= mid).astype(jnp.int32), axis=0, keepdims=True)
    take_lo = cnt >= (_K + 1)
    hi = jnp.where(take_lo, mid, hi)
    lo = jnp.where(take_lo, lo, mid + 1)
    return lo, hi


def _entropy_body(x0_ref, x1_ref, x2_ref, out_ref, hsum_ref):
    b = pl.program_id(0)
    tri = [_bits_and_bracket(ref[0]) for ref in (x0_ref, x1_ref, x2_ref)]
    bits3 = [t[0] for t in tri]

    def cond(carry):
        los, his = carry
        return (jnp.any(los[0] < his[0]) | jnp.any(los[1] < his[1])
                | jnp.any(los[2] < his[2]))

    def step(carry):
        los, his = carry
        for _ in range(2):
            new = [_one_step(bits3[k], los[k], his[k]) for k in range(3)]
            los = [n[0] for n in new]
            his = [n[1] for n in new]
        return los, his

    los0 = [t[1] for t in tri]
    his0 = [t[2] for t in tri]
    los, _ = jax.lax.while_loop(cond, step, (los0, his0))

    for k in range(3):
        kth = jax.lax.bitcast_convert_type(los[k], jnp.float32)
        s = jnp.sum(jnp.sqrt(kth))

        @pl.when(b == 0)
        def _(k=k, s=s):
            hsum_ref[k] = s

        @pl.when(b != 0)
        def _(k=k, s=s):
            hsum_ref[k] = hsum_ref[k] + s

    @pl.when(b == pl.num_programs(0) - 1)
    def _():
        e0 = jnp.log(jnp.full((1, 128), hsum_ref[0]) + 1.0)
        e1 = jnp.log(jnp.full((1, 128), hsum_ref[1]) + 1.0)
        e2 = jnp.log(jnp.full((1, 128), hsum_ref[2]) + 1.0)
        d0 = e1 - e0
        d1 = e2 - e1
        out_ref[0] = (d0 - d1) * (d0 - d1) * 0.5


@functools.partial(jax.jit, static_argnums=())
def kernel(feat0, feat1, feat2):
    B, C, H, W = feat0.shape
    xs = [f.reshape(B, C, H * W) for f in (feat0, feat1, feat2)]
    out = pl.pallas_call(
        _entropy_body,
        grid=(B,),
        in_specs=[pl.BlockSpec((1, C, H * W), lambda b: (b, 0, 0))] * 3,
        out_specs=pl.BlockSpec((1, 1, 128), lambda b: (0, 0, 0)),
        out_shape=jax.ShapeDtypeStruct((1, 1, 128), jnp.float32),
        scratch_shapes=[pltpu.SMEM((4,), jnp.float32)],
        compiler_params=pltpu.CompilerParams(
            dimension_semantics=("arbitrary",)),
    )(*xs)
    return out[0, 0, 0]
